# Initial kernel scaffold; baseline (speedup 1.0000x reference)
import jax
import jax.numpy as jnp
from jax.experimental import pallas as pl


def kernel(embedding_features_per_residue, edge_index, edge_attr, batch, embedding_features_per_sequence, W1, b1, W2, b2, W3, b3, g1, be1, g2, be2, g3, be3, fc1_W, fc1_b, lin_W, lin_b):
    raise NotImplementedError("write your pallas kernel here")


# speedup vs baseline: 7.6337x; 7.6337x over previous
import functools

import jax
import jax.numpy as jnp
from jax import lax
from jax.experimental import pallas as pl
from jax.experimental.pallas import tpu as pltpu
from jax.experimental.pallas import tpu_sc as plsc

N = 10000
NPAD = 10240
E = 320000
D = 128
NB = 16
NCLS = 10

NC = 2
NS = 16
NW = NC * NS
EPW = E // NW
CH = 80
NCHUNK = EPW // CH
RPT = NPAD // NS
WB = 128


@functools.lru_cache(maxsize=None)
def _build_deg_kernel():
    mesh = plsc.VectorSubcoreMesh(
        core_axis_name="c", subcore_axis_name="s", num_cores=NC, num_subcores=NS
    )
    return functools.partial(
        pl.kernel,
        out_type=jax.ShapeDtypeStruct((NC, NPAD), jnp.float32),
        mesh=mesh,
        scratch_types=[
            pltpu.VMEM_SHARED((NPAD,), jnp.float32),
            pltpu.VMEM((CH,), jnp.int32),
            pltpu.VMEM((CH,), jnp.float32),
            pltpu.VMEM((RPT,), jnp.float32),
        ],
        compiler_params=pltpu.CompilerParams(needs_layout_passes=False),
    )(_deg_body)


def _deg_body(dst_hbm, ew_hbm, zrow_hbm, out_hbm, acc_sh, idx_v, val_v, row_v):
    cid = lax.axis_index("c")
    sid = lax.axis_index("s")
    wid = sid * NC + cid

    pltpu.sync_copy(zrow_hbm.at[pl.ds(sid * RPT, RPT)], row_v)
    pltpu.sync_copy(row_v, acc_sh.at[pl.ds(sid * RPT, RPT)])
    plsc.subcore_barrier()

    def chunk(c, carry):
        base = wid * EPW + c * CH
        pltpu.sync_copy(dst_hbm.at[pl.ds(base, CH)], idx_v)
        pltpu.sync_copy(ew_hbm.at[pl.ds(base, CH)], val_v)
        pltpu.sync_copy(val_v, acc_sh.at[idx_v], add=True)
        return carry

    lax.fori_loop(0, NCHUNK, chunk, 0)
    plsc.subcore_barrier()

    pltpu.sync_copy(acc_sh.at[pl.ds(sid * RPT, RPT)], row_v)
    pltpu.sync_copy(row_v, out_hbm.at[cid, pl.ds(sid * RPT, RPT)])


@functools.lru_cache(maxsize=None)
def _build_agg_kernel():
    mesh = plsc.VectorSubcoreMesh(
        core_axis_name="c", subcore_axis_name="s", num_cores=NC, num_subcores=NS
    )
    return functools.partial(
        pl.kernel,
        out_type=jax.ShapeDtypeStruct((NC, NPAD, D), jnp.float32),
        mesh=mesh,
        scratch_types=[
            pltpu.VMEM_SHARED((NPAD, D), jnp.float32),
            pltpu.VMEM((CH,), jnp.int32),
            pltpu.VMEM((CH,), jnp.int32),
            pltpu.VMEM((CH,), jnp.float32),
            pltpu.VMEM((CH, D), jnp.float32),
            pltpu.VMEM((WB, D), jnp.float32),
            pltpu.SemaphoreType.DMA,
        ],
        compiler_params=pltpu.CompilerParams(needs_layout_passes=False),
    )(_agg_body)


def _agg_body(hs_hbm, src_hbm, dst_hbm, ew_hbm, zrows_hbm, out_hbm,
              acc_sh, src_v, dst_v, ew_v, rows_v, bounce_v, sem):
    cid = lax.axis_index("c")
    sid = lax.axis_index("s")
    wid = sid * NC + cid

    pltpu.sync_copy(zrows_hbm, bounce_v)
    for r in range(RPT // WB):
        pltpu.sync_copy(bounce_v, acc_sh.at[pl.ds(sid * RPT + r * WB, WB)])
    plsc.subcore_barrier()

    def chunk(c, carry):
        base = wid * EPW + c * CH
        pltpu.sync_copy(src_hbm.at[pl.ds(base, CH)], src_v)
        pltpu.sync_copy(dst_hbm.at[pl.ds(base, CH)], dst_v)
        pltpu.sync_copy(ew_hbm.at[pl.ds(base, CH)], ew_v)
        pltpu.async_copy(hs_hbm.at[src_v], rows_v, sem).wait()

        def edge(e, ecarry):
            ewb = plsc.load_gather(ew_v, [jnp.full((16,), e, jnp.int32)])
            for j in range(D // 16):
                sl = rows_v[e, pl.ds(j * 16, 16)]
                rows_v[e, pl.ds(j * 16, 16)] = sl * ewb
            return ecarry

        lax.fori_loop(0, CH, edge, 0)
        pltpu.sync_copy(rows_v, acc_sh.at[dst_v], add=True)
        return carry

    lax.fori_loop(0, NCHUNK, chunk, 0)
    plsc.subcore_barrier()

    for r in range(RPT // WB):
        start = sid * RPT + r * WB
        pltpu.sync_copy(acc_sh.at[pl.ds(start, WB)], bounce_v)
        pltpu.sync_copy(bounce_v, out_hbm.at[cid, pl.ds(start, WB)])


def _dis_body(degp_ref, dis_ref):
    deg = degp_ref[0:1, :] + degp_ref[1:2, :] + 1.0
    dis_ref[...] = lax.rsqrt(deg)


def _scale_matmul_body(x_ref, w_ref, dis_ref, out_ref):
    h = jnp.dot(x_ref[...], w_ref[...], preferred_element_type=jnp.float32)
    out_ref[...] = h * dis_ref[...]


def _post_body(aggp_ref, hs_ref, dis_ref, b_ref, g_ref, be_ref, w_ref, out_ref):
    s = (aggp_ref[0, 0:N, :] + aggp_ref[1, 0:N, :] + hs_ref[...]) * dis_ref[...]
    t = jnp.maximum(s + b_ref[...], 0.0)
    mu = jnp.mean(t, axis=0, keepdims=True)
    var = jnp.mean((t - mu) ** 2, axis=0, keepdims=True)
    xn = (t - mu) * lax.rsqrt(var + 1e-5) * g_ref[...] + be_ref[...]
    out_ref[...] = (
        jnp.dot(xn, w_ref[...], preferred_element_type=jnp.float32) * dis_ref[...]
    )


def _head_body(aggp_ref, hs_ref, dis_ref, b_ref, g_ref, be_ref, batch_ref,
               seq_ref, fc1w_ref, fc1b_ref, linw_ref, linb_ref, out_ref):
    s = (aggp_ref[0, 0:N, :] + aggp_ref[1, 0:N, :] + hs_ref[...]) * dis_ref[...]
    t = s + b_ref[...]
    mu = jnp.mean(t, axis=0, keepdims=True)
    var = jnp.mean((t - mu) ** 2, axis=0, keepdims=True)
    xn = (t - mu) * lax.rsqrt(var + 1e-5) * g_ref[...] + be_ref[...]
    seg = lax.broadcasted_iota(jnp.int32, (NB, N), 0)
    onehot = (batch_ref[...] == seg).astype(jnp.float32)
    psum = jnp.dot(onehot, xn, preferred_element_type=jnp.float32)
    cnt = jnp.dot(onehot, jnp.ones((N, 1), jnp.float32),
                  preferred_element_type=jnp.float32)
    pooled = psum / jnp.maximum(cnt, 1.0)
    y = jnp.dot(seq_ref[...], fc1w_ref[...],
                preferred_element_type=jnp.float32) + fc1b_ref[...]
    z = pooled + y
    logits = jnp.dot(z, linw_ref[...],
                     preferred_element_type=jnp.float32) + linb_ref[...]
    out_ref[...] = jax.nn.sigmoid(logits)


def _tc_call(body, out_shape, *args):
    return pl.pallas_call(body, out_shape=out_shape)(*args)


def kernel(embedding_features_per_residue, edge_index, edge_attr, batch,
           embedding_features_per_sequence, W1, b1, W2, b2, W3, b3,
           g1, be1, g2, be2, g3, be3, fc1_W, fc1_b, lin_W, lin_b):
    x = embedding_features_per_residue
    src = edge_index[0]
    dst = edge_index[1]
    ew = edge_attr[:, 0]
    zrow = jnp.zeros((NPAD,), jnp.float32)
    zrows = jnp.zeros((WB, D), jnp.float32)

    degp = _build_deg_kernel()(dst, ew, zrow)
    dis_row = _tc_call(_dis_body,
                       jax.ShapeDtypeStruct((1, NPAD), jnp.float32), degp)
    dis = dis_row.reshape(NPAD, 1)[:N]

    hs = _tc_call(_scale_matmul_body,
                  jax.ShapeDtypeStruct((N, D), jnp.float32), x, W1, dis)

    for (b, g, be, Wn) in ((b1, g1, be1, W2), (b2, g2, be2, W3)):
        aggp = _build_agg_kernel()(hs, src, dst, ew, zrows)
        hs = _tc_call(_post_body,
                      jax.ShapeDtypeStruct((N, D), jnp.float32),
                      aggp, hs, dis, b.reshape(1, D), g.reshape(1, D),
                      be.reshape(1, D), Wn)

    aggp = _build_agg_kernel()(hs, src, dst, ew, zrows)
    out = _tc_call(_head_body,
                   jax.ShapeDtypeStruct((NB, NCLS), jnp.float32),
                   aggp, hs, dis, b3.reshape(1, D), g3.reshape(1, D),
                   be3.reshape(1, D), batch.reshape(1, N),
                   embedding_features_per_sequence, fc1_W,
                   fc1_b.reshape(1, D), lin_W, lin_b.reshape(1, NCLS))
    return out


# speedup vs baseline: 19.1358x; 2.5068x over previous
import functools

import jax
import jax.numpy as jnp
from jax import lax
from jax.experimental import pallas as pl
from jax.experimental.pallas import tpu as pltpu
from jax.experimental.pallas import tpu_sc as plsc

N = 10000
NPAD = 10240
E = 320000
D = 128
NB = 16
NCLS = 10

NC = 2
NS = 16
NW = NC * NS
EPW = E // NW
CH = 80
TPC = EPW // CH
RPT = NPAD // NS
WB = 128
NBUF = 2
RPN = N // NS


@functools.lru_cache(maxsize=None)
def _build_deg_kernel():
    mesh = plsc.VectorSubcoreMesh(
        core_axis_name="c", subcore_axis_name="s", num_cores=NC, num_subcores=NS
    )
    return functools.partial(
        pl.kernel,
        out_type=jax.ShapeDtypeStruct((NC, NPAD), jnp.float32),
        mesh=mesh,
        scratch_types=[
            pltpu.VMEM_SHARED((NPAD,), jnp.float32),
            pltpu.VMEM((TPC, CH), jnp.int32),
            pltpu.VMEM((TPC, CH), jnp.float32),
            pltpu.VMEM((RPT,), jnp.float32),
        ],
        compiler_params=pltpu.CompilerParams(needs_layout_passes=False, use_tc_tiling_on_sc=False),
    )(_deg_body)


def _deg_body(dst2d_hbm, ew2d_hbm, zrow_hbm, out_hbm, acc_sh, dsts_v, ews_v, row_v):
    cid = lax.axis_index("c")
    sid = lax.axis_index("s")
    wid = sid * NC + cid

    pltpu.sync_copy(dst2d_hbm.at[pl.ds(wid * TPC, TPC)], dsts_v)
    pltpu.sync_copy(ew2d_hbm.at[pl.ds(wid * TPC, TPC)], ews_v)
    pltpu.sync_copy(zrow_hbm.at[pl.ds(sid * RPT, RPT)], row_v)
    pltpu.sync_copy(row_v, acc_sh.at[pl.ds(sid * RPT, RPT)])
    plsc.subcore_barrier()

    def chunk(c, carry):
        pltpu.sync_copy(ews_v.at[c], acc_sh.at[dsts_v.at[c]], add=True)
        return carry

    lax.fori_loop(0, TPC, chunk, 0)
    plsc.subcore_barrier()

    pltpu.sync_copy(acc_sh.at[pl.ds(sid * RPT, RPT)], row_v)
    pltpu.sync_copy(row_v, out_hbm.at[cid, pl.ds(sid * RPT, RPT)])


@functools.lru_cache(maxsize=None)
def _build_agg_kernel():
    mesh = plsc.VectorSubcoreMesh(
        core_axis_name="c", subcore_axis_name="s", num_cores=NC, num_subcores=NS
    )
    return functools.partial(
        pl.kernel,
        out_type=jax.ShapeDtypeStruct((NC, N, D), jnp.float32),
        mesh=mesh,
        scratch_types=[
            pltpu.VMEM_SHARED((N, D), jnp.float32),
            pltpu.VMEM((TPC, CH), jnp.int32),
            pltpu.VMEM((TPC, CH), jnp.int32),
            pltpu.VMEM((TPC, CH), jnp.float32),
            [pltpu.VMEM((CH, D), jnp.float32)] * NBUF,
            [pltpu.SemaphoreType.DMA] * NBUF,
            [pltpu.SemaphoreType.DMA] * NBUF,
        ],
        compiler_params=pltpu.CompilerParams(needs_layout_passes=False, use_tc_tiling_on_sc=False),
    )(_agg_body)


def _agg_body(hs_hbm, src2d_hbm, dst2d_hbm, ew2d_hbm, zrows_hbm, out_hbm,
              acc_sh, srcs_v, dsts_v, ews_v, rows, gsems, ssems):
    cid = lax.axis_index("c")
    sid = lax.axis_index("s")
    wid = sid * NC + cid

    pltpu.sync_copy(src2d_hbm.at[pl.ds(wid * TPC, TPC)], srcs_v)
    pltpu.sync_copy(dst2d_hbm.at[pl.ds(wid * TPC, TPC)], dsts_v)
    pltpu.sync_copy(ew2d_hbm.at[pl.ds(wid * TPC, TPC)], ews_v)
    pltpu.sync_copy(zrows_hbm, rows[0])
    for r in range(RPN // CH):
        pltpu.sync_copy(rows[0], acc_sh.at[pl.ds(sid * RPN + r * CH, CH)])
    rtail = RPN % CH
    pltpu.sync_copy(rows[0].at[pl.ds(0, rtail)],
                    acc_sh.at[pl.ds(sid * RPN + (RPN // CH) * CH, rtail)])
    plsc.subcore_barrier()

    def gather_start(c, k):
        pltpu.async_copy(hs_hbm.at[srcs_v.at[c]], rows[k], gsems[k])

    def gather_wait(c, k):
        pltpu.make_async_copy(hs_hbm.at[srcs_v.at[c]], rows[k], gsems[k]).wait()

    def scatter_start(c, k):
        pltpu.async_copy(rows[k], acc_sh.at[dsts_v.at[c]], ssems[k], add=True)

    def scatter_wait(c, k):
        pltpu.make_async_copy(rows[k], acc_sh.at[dsts_v.at[c]], ssems[k]).wait()

    def mul_chunk(c, k):
        def edge(e, ecarry):
            ewb = plsc.load_gather(
                ews_v, [jnp.full((16,), c, jnp.int32), jnp.full((16,), e, jnp.int32)]
            )
            for j in range(D // 16):
                sl = rows[k][e, pl.ds(j * 16, 16)]
                rows[k][e, pl.ds(j * 16, 16)] = sl * ewb
            return ecarry

        lax.fori_loop(0, CH, edge, 0)

    gather_start(0, 0)
    npair = TPC // NBUF

    def ring(p, carry):
        for k in range(NBUF):
            c = NBUF * p + k
            gather_wait(c, k)
            kn = (k + 1) % NBUF
            if k == NBUF - 1:
                scatter_wait(NBUF * p, kn)
                gather_start(c + 1, kn)
            else:
                @pl.when(p > 0)
                def _():
                    scatter_wait(c + 1 - NBUF, kn)
                gather_start(c + 1, kn)
            mul_chunk(c, k)
            scatter_start(c, k)
        return carry

    lax.fori_loop(0, npair, ring, 0)
    ct = TPC - 1
    gather_wait(ct, 0)
    mul_chunk(ct, 0)
    scatter_start(ct, 0)
    scatter_wait(ct, 0)
    for k in range(1, NBUF):
        scatter_wait(ct - NBUF + k, k)
    plsc.subcore_barrier()

    for r in range(RPN // CH):
        start = sid * RPN + r * CH
        pltpu.sync_copy(acc_sh.at[pl.ds(start, CH)], rows[r % NBUF])
        pltpu.sync_copy(rows[r % NBUF], out_hbm.at[cid, pl.ds(start, CH)])
    start = sid * RPN + (RPN // CH) * CH
    pltpu.sync_copy(acc_sh.at[pl.ds(start, rtail)], rows[0].at[pl.ds(0, rtail)])
    pltpu.sync_copy(rows[0].at[pl.ds(0, rtail)], out_hbm.at[cid, pl.ds(start, rtail)])


def _dis_body(degp_ref, dis_ref):
    deg = degp_ref[0:1, :] + degp_ref[1:2, :] + 1.0
    dis_ref[...] = lax.rsqrt(deg)


def _scale_matmul_body(x_ref, w_ref, dis_ref, out_ref):
    h = jnp.dot(x_ref[...], w_ref[...], preferred_element_type=jnp.float32)
    out_ref[...] = h * dis_ref[...]


def _post_body(aggp_ref, hs_ref, dis_ref, b_ref, g_ref, be_ref, w_ref, out_ref):
    s = (aggp_ref[0, 0:N, :] + aggp_ref[1, 0:N, :] + hs_ref[...]) * dis_ref[...]
    t = jnp.maximum(s + b_ref[...], 0.0)
    mu = jnp.mean(t, axis=0, keepdims=True)
    var = jnp.mean((t - mu) ** 2, axis=0, keepdims=True)
    xn = (t - mu) * lax.rsqrt(var + 1e-5) * g_ref[...] + be_ref[...]
    out_ref[...] = (
        jnp.dot(xn, w_ref[...], preferred_element_type=jnp.float32) * dis_ref[...]
    )


def _head_body(aggp_ref, hs_ref, dis_ref, b_ref, g_ref, be_ref, batch_ref,
               seq_ref, fc1w_ref, fc1b_ref, linw_ref, linb_ref, out_ref):
    s = (aggp_ref[0, 0:N, :] + aggp_ref[1, 0:N, :] + hs_ref[...]) * dis_ref[...]
    t = s + b_ref[...]
    mu = jnp.mean(t, axis=0, keepdims=True)
    var = jnp.mean((t - mu) ** 2, axis=0, keepdims=True)
    xn = (t - mu) * lax.rsqrt(var + 1e-5) * g_ref[...] + be_ref[...]
    seg = lax.broadcasted_iota(jnp.int32, (NB, N), 0)
    onehot = (batch_ref[...] == seg).astype(jnp.float32)
    psum = jnp.dot(onehot, xn, preferred_element_type=jnp.float32)
    cnt = jnp.dot(onehot, jnp.ones((N, 1), jnp.float32),
                  preferred_element_type=jnp.float32)
    pooled = psum / jnp.maximum(cnt, 1.0)
    y = jnp.dot(seq_ref[...], fc1w_ref[...],
                preferred_element_type=jnp.float32) + fc1b_ref[...]
    z = pooled + y
    logits = jnp.dot(z, linw_ref[...],
                     preferred_element_type=jnp.float32) + linb_ref[...]
    out_ref[...] = jax.nn.sigmoid(logits)


def _tc_call(body, out_shape, *args):
    return pl.pallas_call(body, out_shape=out_shape)(*args)


def kernel(embedding_features_per_residue, edge_index, edge_attr, batch,
           embedding_features_per_sequence, W1, b1, W2, b2, W3, b3,
           g1, be1, g2, be2, g3, be3, fc1_W, fc1_b, lin_W, lin_b):
    x = embedding_features_per_residue
    src = edge_index[0].reshape(E // CH, CH)
    dst = edge_index[1].reshape(E // CH, CH)
    ew = edge_attr[:, 0].reshape(E // CH, CH)
    zrow = jnp.zeros((NPAD,), jnp.float32)
    zrows = jnp.zeros((CH, D), jnp.float32)

    degp = _build_deg_kernel()(dst, ew, zrow)
    dis_row = _tc_call(_dis_body,
                       jax.ShapeDtypeStruct((1, NPAD), jnp.float32), degp)
    dis = dis_row.reshape(NPAD, 1)[:N]

    hs = _tc_call(_scale_matmul_body,
                  jax.ShapeDtypeStruct((N, D), jnp.float32), x, W1, dis)

    for (b, g, be, Wn) in ((b1, g1, be1, W2), (b2, g2, be2, W3)):
        aggp = _build_agg_kernel()(hs, src, dst, ew, zrows)
        hs = _tc_call(_post_body,
                      jax.ShapeDtypeStruct((N, D), jnp.float32),
                      aggp, hs, dis, b.reshape(1, D), g.reshape(1, D),
                      be.reshape(1, D), Wn)

    aggp = _build_agg_kernel()(hs, src, dst, ew, zrows)
    out = _tc_call(_head_body,
                   jax.ShapeDtypeStruct((NB, NCLS), jnp.float32),
                   aggp, hs, dis, b3.reshape(1, D), g3.reshape(1, D),
                   be3.reshape(1, D), batch.reshape(1, N),
                   embedding_features_per_sequence, fc1_W,
                   fc1_b.reshape(1, D), lin_W, lin_b.reshape(1, NCLS))
    return out
